# baseline (speedup 1.0000x reference)
import dataclasses

import jax
import jax.numpy as jnp
from jax import lax
from jax.experimental import pallas as pl
from jax.experimental.pallas import tpu as pltpu
from jax.experimental.pallas import tpu_sc as plsc

_NW = 16
_LANES = 16
_RB = 2


def _make_body(B, dim, HW, M):
    PP = B * M // _NW
    EPW = PP * dim
    SB = dim * HW
    BPW = B // _NW
    NR = BPW // _RB
    RPP = _RB * M
    RNCH = RPP * dim // _LANES
    assert dim == 2 and BPW == NR * _RB and RPP * dim % _LANES == 0

    def body(flat_ref, ind_ref, tgt_ref, msk_ref, out_ref,
             ind_v, msk_v, feat_v, tgt_v, acc_v, all_v, res_v, shared,
             sem_f, sem_t, sem_m):
        wid = lax.axis_index("s")
        tgt_cp = pltpu.async_copy(
            tgt_ref.at[pl.ds(wid * EPW, EPW)], tgt_v, sem_t)
        msk_cp = pltpu.async_copy(
            msk_ref.at[pl.ds(wid * PP, PP)], msk_v, sem_m)
        pltpu.sync_copy(ind_ref.at[pl.ds(wid * PP, PP)], ind_v)

        iota = lax.broadcasted_iota(jnp.int32, (_LANES,), 0)
        half = lax.shift_right_logical(iota, 1)
        d_off = (iota & 1) * HW
        tgt_cp.wait()
        msk_cp.wait()

        acc_l = jnp.zeros((_LANES,), jnp.float32)
        acc_m = jnp.zeros((_LANES,), jnp.float32)
        for r in range(NR):
            pltpu.async_copy(
                flat_ref.at[pl.ds((wid * BPW + r * _RB) * SB, _RB * SB)],
                feat_v, sem_f).wait()
            for c in range(RNCH):
                p_rel = half + (c * (_LANES // dim))
                p_loc = p_rel + r * RPP
                ind_g = plsc.load_gather(ind_v, [p_loc])
                b_off = jnp.where(p_rel >= M, SB, 0)
                v = plsc.load_gather(feat_v, [b_off + d_off + ind_g])
                t = tgt_v[pl.ds((r * RNCH + c) * _LANES, _LANES)]
                m = plsc.load_gather(msk_v, [p_loc]).astype(jnp.float32)
                a = jnp.abs(v - t) * m
                acc_l = acc_l + jnp.where(a < 1.0, 0.5 * a * a, a - 0.5)
                acc_m = acc_m + m
        acc_v[0, :] = acc_l
        acc_v[1, :] = acc_m

        pltpu.sync_copy(acc_v, shared.at[wid])
        plsc.subcore_barrier()

        @pl.when(wid == 0)
        def _():
            pltpu.sync_copy(shared, all_v)
            al = jnp.zeros((_LANES,), jnp.float32)
            am = jnp.zeros((_LANES,), jnp.float32)
            for i in range(_NW):
                al = al + all_v[i, 0, :]
                am = am + all_v[i, 1, :]
            lv = jnp.broadcast_to(jnp.sum(al), (_LANES,))
            nv = jnp.broadcast_to(jnp.sum(am), (_LANES,))
            d = nv / dim + 1e-4
            r = 1.0 / d
            r = r * (2.0 - d * r)
            res_v[...] = lv * r
            pltpu.sync_copy(res_v.at[pl.ds(0, 1)], out_ref)

    return body


def kernel(output, mask, ind, target):
    B, dim, H, W = output.shape
    M = ind.shape[1]
    HW = H * W
    PP = B * M // _NW
    EPW = PP * dim

    cp = pltpu.CompilerParams()
    if "needs_layout_passes" in pltpu.CompilerParams.__dataclass_fields__:
        cp = dataclasses.replace(cp, needs_layout_passes=False)
    mesh = plsc.VectorSubcoreMesh(
        core_axis_name="c", subcore_axis_name="s", num_cores=1)
    fn = pl.kernel(
        _make_body(B, dim, HW, M),
        out_type=jax.ShapeDtypeStruct((1,), jnp.float32),
        mesh=mesh,
        compiler_params=cp,
        scratch_types=[
            pltpu.VMEM((PP,), jnp.int32),
            pltpu.VMEM((PP,), jnp.int32),
            pltpu.VMEM((_RB * dim * HW,), jnp.float32),
            pltpu.VMEM((EPW,), jnp.float32),
            pltpu.VMEM((2, _LANES), jnp.float32),
            pltpu.VMEM((_NW, 2, _LANES), jnp.float32),
            pltpu.VMEM((_LANES,), jnp.float32),
            pltpu.VMEM_SHARED((_NW, 2, _LANES), jnp.float32),
            pltpu.SemaphoreType.DMA,
            pltpu.SemaphoreType.DMA,
            pltpu.SemaphoreType.DMA,
        ],
    )
    parts = fn(output.reshape(-1), ind.reshape(-1), target.reshape(-1),
               mask.reshape(-1))
    return parts.reshape(())
